# baseline (speedup 1.0000x reference)
import functools

import jax
import jax.numpy as jnp
from jax import lax
from jax.experimental import pallas as pl
from jax.experimental.pallas import tpu as pltpu
from jax.experimental.pallas import tpu_sc as plsc

_D = 128


def _sc_copy_pair(rows):
    info = plsc.get_sparse_core_info()
    nc, ns = info.num_cores, info.num_subcores
    nw = nc * ns
    rpw = rows // nw
    mesh = plsc.VectorSubcoreMesh(core_axis_name="c", subcore_axis_name="s")

    @functools.partial(
        pl.kernel,
        mesh=mesh,
        out_type=(
            jax.ShapeDtypeStruct((rows, _D), jnp.float32),
            jax.ShapeDtypeStruct((rows, _D), jnp.float32),
        ),
        scratch_types=[pltpu.SemaphoreType.DMA, pltpu.SemaphoreType.DMA],
    )
    def sc_copy(k_hbm, v_hbm, ko_hbm, vo_hbm, sem_k, sem_v):
        wid = lax.axis_index("s") * nc + lax.axis_index("c")
        base = wid * rpw
        ck = pltpu.async_copy(
            k_hbm.at[pl.ds(base, rpw)], ko_hbm.at[pl.ds(base, rpw)], sem_k
        )
        cv = pltpu.async_copy(
            v_hbm.at[pl.ds(base, rpw)], vo_hbm.at[pl.ds(base, rpw)], sem_v
        )
        ck.wait()
        cv.wait()

    return sc_copy


def kernel(key_states, value_states, layer_idx):
    shape = key_states.shape
    rows = shape[0] * shape[1] * shape[2]
    k2 = key_states.reshape(rows, _D)
    v2 = value_states.reshape(rows, _D)
    ko, vo = _sc_copy_pair(rows)(k2, v2)
    return (ko.reshape(shape), vo.reshape(shape))


# speedup vs baseline: 38.9157x; 38.9157x over previous
import functools

import jax
import jax.numpy as jnp
from jax import lax
from jax.experimental import pallas as pl
from jax.experimental.pallas import tpu as pltpu
from jax.experimental.pallas import tpu_sc as plsc

_D = 128
_C = 256
_NB = 3


def _sc_copy_pair(rows):
    info = plsc.get_sparse_core_info()
    nc, ns = info.num_cores, info.num_subcores
    nw = nc * ns
    rpw = rows // nw
    cpt = rpw // _C
    n = 2 * cpt

    mesh = plsc.VectorSubcoreMesh(core_axis_name="c", subcore_axis_name="s")

    @functools.partial(
        pl.kernel,
        mesh=mesh,
        out_type=(
            jax.ShapeDtypeStruct((rows, _D), jnp.float32),
            jax.ShapeDtypeStruct((rows, _D), jnp.float32),
        ),
        scratch_types=(
            [pltpu.VMEM((_C, _D), jnp.float32) for _ in range(_NB)]
            + [pltpu.SemaphoreType.DMA for _ in range(2 * _NB)]
        ),
    )
    def sc_copy(k_hbm, v_hbm, ko_hbm, vo_hbm, *scratch):
        bufs = scratch[:_NB]
        sin = scratch[_NB : 2 * _NB]
        sout = scratch[2 * _NB :]
        wid = lax.axis_index("s") * nc + lax.axis_index("c")
        base = wid * rpw

        def src_dst_off(i):
            if i < cpt:
                return k_hbm, ko_hbm, base + i * _C
            return v_hbm, vo_hbm, base + (i - cpt) * _C

        in_copies = [None] * n
        out_copies = [None] * n
        for i in range(min(_NB, n)):
            src, _, off = src_dst_off(i)
            in_copies[i] = pltpu.async_copy(
                src.at[pl.ds(off, _C)], bufs[i % _NB], sin[i % _NB]
            )
        for i in range(n):
            b = i % _NB
            if i >= _NB:
                out_copies[i - _NB].wait()
                src, _, off = src_dst_off(i)
                in_copies[i] = pltpu.async_copy(
                    src.at[pl.ds(off, _C)], bufs[b], sin[b]
                )
            in_copies[i].wait()
            _, dst, off = src_dst_off(i)
            out_copies[i] = pltpu.async_copy(
                bufs[b], dst.at[pl.ds(off, _C)], sout[b]
            )
        for i in range(max(0, n - _NB), n):
            out_copies[i].wait()

    return sc_copy


def kernel(key_states, value_states, layer_idx):
    shape = key_states.shape
    rows = shape[0] * shape[1] * shape[2]
    k2 = key_states.reshape(rows, _D)
    v2 = value_states.reshape(rows, _D)
    ko, vo = _sc_copy_pair(rows)(k2, v2)
    return (ko.reshape(shape), vo.reshape(shape))


# speedup vs baseline: 42.0943x; 1.0817x over previous
import functools

import jax
import jax.numpy as jnp
from jax import lax
from jax.experimental import pallas as pl
from jax.experimental.pallas import tpu as pltpu
from jax.experimental.pallas import tpu_sc as plsc

_D = 128
_C = 256
_NB = 3


def _sc_copy_one(rows):
    info = plsc.get_sparse_core_info()
    nc, ns = info.num_cores, info.num_subcores
    nw = nc * ns
    rpw = rows // nw
    n = rpw // _C

    mesh = plsc.VectorSubcoreMesh(core_axis_name="c", subcore_axis_name="s")

    @functools.partial(
        pl.kernel,
        mesh=mesh,
        out_type=jax.ShapeDtypeStruct((rows, _D), jnp.float32),
        scratch_types=(
            [pltpu.VMEM((_C, _D), jnp.float32) for _ in range(_NB)]
            + [pltpu.SemaphoreType.DMA for _ in range(2 * _NB)]
        ),
    )
    def sc_copy(src_hbm, dst_hbm, *scratch):
        bufs = scratch[:_NB]
        sin = scratch[_NB : 2 * _NB]
        sout = scratch[2 * _NB :]
        wid = lax.axis_index("s") * nc + lax.axis_index("c")
        base = wid * rpw

        in_copies = [None] * n
        out_copies = [None] * n
        for i in range(min(_NB, n)):
            in_copies[i] = pltpu.async_copy(
                src_hbm.at[pl.ds(base + i * _C, _C)], bufs[i % _NB], sin[i % _NB]
            )
        for i in range(n):
            b = i % _NB
            if i >= _NB:
                out_copies[i - _NB].wait()
                in_copies[i] = pltpu.async_copy(
                    src_hbm.at[pl.ds(base + i * _C, _C)], bufs[b], sin[b]
                )
            in_copies[i].wait()
            out_copies[i] = pltpu.async_copy(
                bufs[b], dst_hbm.at[pl.ds(base + i * _C, _C)], sout[b]
            )
        for i in range(max(0, n - _NB), n):
            out_copies[i].wait()

    return sc_copy


def _tc_copy_body(in_ref, out_ref):
    out_ref[...] = in_ref[...]


def _tc_copy_one(rows):
    blk = 8192
    spec = pl.BlockSpec((blk, _D), lambda i: (i, 0))
    return pl.pallas_call(
        _tc_copy_body,
        grid=(rows // blk,),
        out_shape=jax.ShapeDtypeStruct((rows, _D), jnp.float32),
        in_specs=[spec],
        out_specs=spec,
    )


def kernel(key_states, value_states, layer_idx):
    shape = key_states.shape
    rows = shape[0] * shape[1] * shape[2]
    k2 = key_states.reshape(rows, _D)
    v2 = value_states.reshape(rows, _D)
    ko = _tc_copy_one(rows)(k2)
    vo = _sc_copy_one(rows)(v2)
    return (ko.reshape(shape), vo.reshape(shape))
